# baseline (speedup 1.0000x reference)
import functools

import jax
import jax.numpy as jnp
from jax import lax
from jax.experimental import pallas as pl
from jax.experimental.pallas import tpu as pltpu
from jax.experimental.pallas import tpu_sc as plsc

N1 = 128
D = 1024
DH = 512
CT = 32
NQ = 4
NCHUNK = 32
NBUF = 3


def _sc_body(x_hbm, p1_hbm, p2_hbm, out_hbm,
             b0, b1, b2, p1m, pr0, pr1, pr2,
             si0, si1, si2, so0, so1, so2, sp0, sp1, sp2):
    cid = lax.axis_index("c")
    sid = lax.axis_index("s")
    wid = sid * 2 + cid
    q = wid % NQ
    g = wid // NQ

    pltpu.sync_copy(p1_hbm.at[pl.ds(q * CT, CT)], p1m)

    def row0_of(c):
        return (g * NCHUNK + c) * N1 + q * CT

    def s_of(c):
        return (g * NCHUNK + c) % (N1 // 2)

    bufs = (b0, b1, b2)
    sis = (si0, si1, si2)
    sos = (so0, so1, so2)
    prs = (pr0, pr1, pr2)
    sps = (sp0, sp1, sp2)

    def start_in(slot, c):
        pltpu.make_async_copy(x_hbm.at[pl.ds(row0_of(c), CT)],
                              bufs[slot], sis[slot]).start()

    def wait_in(slot):
        pltpu.make_async_copy(x_hbm.at[pl.ds(0, CT)],
                              bufs[slot], sis[slot]).wait()

    def start_out(slot, c):
        pltpu.make_async_copy(bufs[slot], out_hbm.at[pl.ds(row0_of(c), CT)],
                              sos[slot]).start()

    def wait_out(slot):
        pltpu.make_async_copy(bufs[slot], out_hbm.at[pl.ds(0, CT)],
                              sos[slot]).wait()

    def start_p2(slot, c):
        pltpu.make_async_copy(p2_hbm.at[pl.ds(s_of(c), 1)],
                              prs[slot], sps[slot]).start()

    def wait_p2(slot):
        pltpu.make_async_copy(p2_hbm.at[pl.ds(0, 1)],
                              prs[slot], sps[slot]).wait()

    start_in(0, 0)
    start_p2(0, 0)
    start_in(1, 1)
    start_p2(1, 1)

    def compute_chunk(buf, pr):
        p2v = [pr[0, pl.ds(k * 16, 16)] for k in range(DH // 16)]

        def tok_body(tt, _):
            for u in range(2):
                t = tt * 2 + u
                for k in range(DH // 16):
                    sl = pl.ds(k * 16, 16)
                    plsc.addupdate(buf.at[t, sl], p1m[t, sl])
                for k in range(DH // 16):
                    sl = pl.ds(DH + k * 16, 16)
                    plsc.addupdate(buf.at[t, sl], p2v[k])
            return ()

        lax.fori_loop(0, CT // 2, tok_body, ())

    def chunk_step(c, p):
        nslot = (p + 2) % NBUF
        wait_in(p)
        wait_p2(p)
        compute_chunk(bufs[p], prs[p])
        start_out(p, c)

        @pl.when(c >= 1)
        def _():
            wait_out(nslot)

        @pl.when(c + 2 < NCHUNK)
        def _():
            start_in(nslot, c + 2)
            start_p2(nslot, c + 2)

    def round_body(rr, _):
        for p in range(NBUF):
            chunk_step(NBUF * rr + p, p)
        return ()

    lax.fori_loop(0, NCHUNK // NBUF, round_body, ())
    chunk_step(jnp.int32(30), 0)
    chunk_step(jnp.int32(31), 1)

    wait_out((NCHUNK - 1) % NBUF)


@jax.jit
def kernel(x, params1, params2):
    b, num_tokens, d_in = x.shape
    x2 = x.reshape(b * num_tokens, d_in)
    mesh = plsc.VectorSubcoreMesh(core_axis_name="c", subcore_axis_name="s")
    f = functools.partial(
        pl.kernel,
        mesh=mesh,
        out_type=jax.ShapeDtypeStruct((b * num_tokens, d_in), x.dtype),
        scratch_types=[
            pltpu.VMEM((CT, D), jnp.float32),
            pltpu.VMEM((CT, D), jnp.float32),
            pltpu.VMEM((CT, D), jnp.float32),
            pltpu.VMEM((CT, DH), jnp.float32),
            pltpu.VMEM((1, DH), jnp.float32),
            pltpu.VMEM((1, DH), jnp.float32),
            pltpu.VMEM((1, DH), jnp.float32),
        ] + [pltpu.SemaphoreType.DMA] * 9,
    )(_sc_body)
    out = f(x2, params1, params2)
    return out.reshape(b, num_tokens, d_in)


# speedup vs baseline: 1.0066x; 1.0066x over previous
import functools

import jax
import jax.numpy as jnp
from jax import lax
from jax.experimental import pallas as pl
from jax.experimental.pallas import tpu as pltpu
from jax.experimental.pallas import tpu_sc as plsc

N1 = 128
D = 1024
DH = 512
CT = 32
NQ = 4
NCHUNK = 32
NBUF = 3


def _sc_body(x_hbm, p1_hbm, p2_hbm, out_hbm,
             b0, b1, b2, p1m, pr0, pr1, pr2,
             si0, si1, si2, so0, so1, so2, sp0, sp1, sp2):
    cid = lax.axis_index("c")
    sid = lax.axis_index("s")
    wid = sid * 2 + cid
    q = wid % NQ
    g = wid // NQ

    pltpu.sync_copy(p1_hbm.at[pl.ds(q * CT, CT)], p1m)

    def row0_of(c):
        return (g * NCHUNK + c) * N1 + q * CT

    def s_of(c):
        return (g * NCHUNK + c) % (N1 // 2)

    bufs = (b0, b1, b2)
    sis = (si0, si1, si2)
    sos = (so0, so1, so2)
    prs = (pr0, pr1, pr2)
    sps = (sp0, sp1, sp2)

    def start_in(slot, c):
        pltpu.make_async_copy(x_hbm.at[pl.ds(row0_of(c), CT)],
                              bufs[slot], sis[slot]).start()

    def wait_in(slot):
        pltpu.make_async_copy(x_hbm.at[pl.ds(0, CT)],
                              bufs[slot], sis[slot]).wait()

    def start_out(slot, c):
        pltpu.make_async_copy(bufs[slot], out_hbm.at[pl.ds(row0_of(c), CT)],
                              sos[slot]).start()

    def wait_out(slot):
        pltpu.make_async_copy(bufs[slot], out_hbm.at[pl.ds(0, CT)],
                              sos[slot]).wait()

    def start_p2(slot, c):
        pltpu.make_async_copy(p2_hbm.at[pl.ds(s_of(c), 1)],
                              prs[slot], sps[slot]).start()

    def wait_p2(slot):
        pltpu.make_async_copy(p2_hbm.at[pl.ds(0, 1)],
                              prs[slot], sps[slot]).wait()

    start_in(0, 0)
    start_p2(0, 0)
    start_in(1, 1)
    start_p2(1, 1)

    def compute_chunk(buf, pr):
        p2v = [pr[0, pl.ds(k * 16, 16)] for k in range(DH // 16)]

        def tok_body(t, _):
            for k in range(DH // 16):
                sl = pl.ds(k * 16, 16)
                plsc.addupdate(buf.at[t, sl], p1m[t, sl])
            for k in range(DH // 16):
                sl = pl.ds(DH + k * 16, 16)
                plsc.addupdate(buf.at[t, sl], p2v[k])
            return ()

        lax.fori_loop(0, CT, tok_body, ())

    def chunk_step(c, p):
        nslot = (p + 2) % NBUF
        wait_in(p)
        wait_p2(p)
        compute_chunk(bufs[p], prs[p])
        start_out(p, c)

        @pl.when(c >= 1)
        def _():
            wait_out(nslot)

        @pl.when(c + 2 < NCHUNK)
        def _():
            start_in(nslot, c + 2)
            start_p2(nslot, c + 2)

    def round_body(rr, _):
        for p in range(NBUF):
            chunk_step(NBUF * rr + p, p)
        return ()

    lax.fori_loop(0, NCHUNK // NBUF, round_body, ())
    chunk_step(jnp.int32(30), 0)
    chunk_step(jnp.int32(31), 1)

    wait_out((NCHUNK - 1) % NBUF)


@jax.jit
def kernel(x, params1, params2):
    b, num_tokens, d_in = x.shape
    x2 = x.reshape(b * num_tokens, d_in)
    mesh = plsc.VectorSubcoreMesh(core_axis_name="c", subcore_axis_name="s")
    f = functools.partial(
        pl.kernel,
        mesh=mesh,
        out_type=jax.ShapeDtypeStruct((b * num_tokens, d_in), x.dtype),
        scratch_types=[
            pltpu.VMEM((CT, D), jnp.float32),
            pltpu.VMEM((CT, D), jnp.float32),
            pltpu.VMEM((CT, D), jnp.float32),
            pltpu.VMEM((CT, DH), jnp.float32),
            pltpu.VMEM((1, DH), jnp.float32),
            pltpu.VMEM((1, DH), jnp.float32),
            pltpu.VMEM((1, DH), jnp.float32),
        ] + [pltpu.SemaphoreType.DMA] * 9,
    )(_sc_body)
    out = f(x2, params1, params2)
    return out.reshape(b, num_tokens, d_in)
